# baseline (speedup 1.0000x reference)
import functools

import jax
import jax.numpy as jnp
from jax import lax
from jax.experimental import pallas as pl
from jax.experimental.pallas import tpu as pltpu
from jax.experimental.pallas import tpu_sc as plsc

EPS = 1e-5


def _make_gather_kernel(B, D, num_cores, num_subcores):
    NW = num_cores * num_subcores
    b_per_w = B // NW
    CH = 128
    n_ch = b_per_w // CH
    mesh = plsc.VectorSubcoreMesh(core_axis_name="c", subcore_axis_name="s")

    @functools.partial(
        pl.kernel,
        mesh=mesh,
        compiler_params=pltpu.CompilerParams(use_tc_tiling_on_sc=False),
        out_type=(
            jax.ShapeDtypeStruct((B, D), jnp.float32),
            jax.ShapeDtypeStruct((B, D), jnp.float32),
        ),
        scratch_types=[
            pltpu.VMEM((n_ch, CH), jnp.int32),
            pltpu.VMEM((n_ch, CH), jnp.int32),
            pltpu.VMEM((n_ch, CH, D), jnp.float32),
            pltpu.VMEM((n_ch, CH, D), jnp.float32),
            pltpu.SemaphoreType.DMA,
            pltpu.SemaphoreType.DMA,
        ],
    )
    def gather_k(uid_hbm, mid_hbm, utab_hbm, mtab_hbm, uout_hbm, mout_hbm,
                 uidx_v, midx_v, urows_v, mrows_v, usem, msem):
        wid = lax.axis_index("s") * num_cores + lax.axis_index("c")
        base = wid * b_per_w
        row0 = wid * n_ch
        pltpu.sync_copy(uid_hbm.at[pl.ds(row0, n_ch)], uidx_v)
        pltpu.sync_copy(mid_hbm.at[pl.ds(row0, n_ch)], midx_v)
        copies = []
        for j in range(n_ch):
            copies.append(
                pltpu.async_copy(utab_hbm.at[uidx_v.at[j]], urows_v.at[j], usem))
            copies.append(
                pltpu.async_copy(mtab_hbm.at[midx_v.at[j]], mrows_v.at[j], msem))
        for c in copies:
            c.wait()
        for j in range(n_ch):
            out_sl = pl.ds(base + j * CH, CH)
            pltpu.sync_copy(urows_v.at[j], uout_hbm.at[out_sl])
            pltpu.sync_copy(mrows_v.at[j], mout_hbm.at[out_sl])

    return gather_k


def _mlp_body(u_ref, m_ref, W1_ref, b1_ref, g1_ref, bt1_ref,
              W2_ref, b2_ref, g2_ref, bt2_ref, W3_ref, b3_ref, o_ref):
    c = 1.0 / (1.0 + EPS) ** 0.5
    u = u_ref[...]
    m = m_ref[...]
    D = u.shape[1]
    h = jnp.dot(u, W1_ref[:D, :], preferred_element_type=jnp.float32)
    h += jnp.dot(m, W1_ref[D:, :], preferred_element_type=jnp.float32)
    h = (h + b1_ref[...]) * (g1_ref[...] * c) + bt1_ref[...]
    h = jnp.maximum(h, 0.0)
    h = jnp.dot(h, W2_ref[...], preferred_element_type=jnp.float32)
    h = (h + b2_ref[...]) * (g2_ref[...] * c) + bt2_ref[...]
    h = jnp.maximum(h, 0.0)
    o = jnp.dot(h, W3_ref[...], preferred_element_type=jnp.float32)
    o_ref[...] = o + b3_ref[...]


def kernel(user_ids, movie_ids, user_table, movie_table,
           W1, b1, g1, beta1, W2, b2, g2, beta2, W3, b3):
    B = user_ids.shape[0]
    D = user_table.shape[1]
    H1 = W1.shape[1]
    H2 = W2.shape[1]

    info = plsc.get_sparse_core_info()
    gather_k = _make_gather_kernel(B, D, info.num_cores, info.num_subcores)
    uid2d = user_ids.reshape(-1, 128)
    mid2d = movie_ids.reshape(-1, 128)
    u_emb, m_emb = gather_k(uid2d, mid2d, user_table, movie_table)

    BLK = 2048
    nblk = B // BLK
    row2d = lambda v: v.reshape(1, -1)
    full = lambda shape: pl.BlockSpec(shape, lambda i: (0, 0))

    out = pl.pallas_call(
        _mlp_body,
        grid=(nblk,),
        in_specs=[
            pl.BlockSpec((BLK, D), lambda i: (i, 0)),
            pl.BlockSpec((BLK, D), lambda i: (i, 0)),
            full((2 * D, H1)),
            full((1, H1)), full((1, H1)), full((1, H1)),
            full((H1, H2)),
            full((1, H2)), full((1, H2)), full((1, H2)),
            full((H2, 1)),
            full((1, 1)),
        ],
        out_specs=pl.BlockSpec((BLK, 1), lambda i: (i, 0)),
        out_shape=jax.ShapeDtypeStruct((B, 1), jnp.float32),
    )(u_emb, m_emb, W1, row2d(b1), row2d(g1), row2d(beta1),
      W2, row2d(b2), row2d(g2), row2d(beta2), W3, row2d(b3))
    return out[:, 0]
